# baseline (speedup 1.0000x reference)
import jax
import jax.numpy as jnp
from jax import lax
from jax.experimental import pallas as pl
from jax.experimental.pallas import tpu as pltpu
from jax.experimental.pallas import tpu_sc as plsc

B, S, D = 4, 8192, 1024
NC, NS, L = 2, 16, 16
NW = NC * NS
ROWS_PER_W = S // NW
C = 8
NCHUNK = ROWS_PER_W // C
NBUF = 3

H = C // 2


def _sc_body(x_hbm, pos_hbm, out_hbm, posb, xb, insemsA, insemsB, outsems):
    cid = lax.axis_index("c")
    sid = lax.axis_index("s")
    wid = sid * NC + cid
    s0 = wid * ROWS_PER_W

    def in_descs(j, chunk, h, sems):
        s = s0 + chunk * C + h * H
        r = h * H
        cps = [pltpu.make_async_copy(
            pos_hbm.at[pl.ds(s, H), :], posb.at[j, pl.ds(r, H), :],
            sems.at[j])]
        for b in range(B):
            cps.append(pltpu.make_async_copy(
                x_hbm.at[b, pl.ds(s, H), :], xb.at[j, b, pl.ds(r, H), :],
                sems.at[j]))
        return cps

    def out_descs(j, chunk, h):
        s = s0 + chunk * C + h * H
        r = h * H
        return [pltpu.make_async_copy(
            xb.at[j, b, pl.ds(r, H), :], out_hbm.at[b, pl.ds(s, H), :],
            outsems.at[j])
            for b in range(B)]

    def start_in(j, chunk):
        for cp in in_descs(j, chunk, 0, insemsA):
            cp.start()
        for cp in in_descs(j, chunk, 1, insemsB):
            cp.start()

    def wait_in(j, chunk, h, sems):
        for cp in in_descs(j, chunk, h, sems):
            cp.wait()

    def start_out(j, chunk, h):
        for cp in out_descs(j, chunk, h):
            cp.start()

    def wait_out(j, chunk):
        for h in range(2):
            for cp in out_descs(j, chunk, h):
                cp.wait()

    def compute(j, h):
        for r in range(h * H, h * H + H):
            def col_body(cc, _, r=r):
                base = cc * (4 * L)
                for u in range(4):
                    off = base + u * L
                    p = posb[j, r, pl.ds(off, L)]
                    for b in range(B):
                        plsc.addupdate(xb.at[j, b, r, pl.ds(off, L)], p)
                return 0
            lax.fori_loop(0, D // (4 * L), col_body, 0, unroll=2)

    start_in(0, 0)
    start_in(1, 1)

    def g_body(t, _):
        j = lax.rem(t, NBUF)

        @pl.when(t < NCHUNK)
        def _():
            wait_in(j, t, 0, insemsA)
            compute(j, 0)
            start_out(j, t, 0)
            wait_in(j, t, 1, insemsB)
            compute(j, 1)
            start_out(j, t, 1)

        @pl.when(t >= 1)
        def _():
            wait_out(lax.rem(j + NBUF - 1, NBUF), t - 1)

        @pl.when(t + 2 < NCHUNK)
        def _():
            start_in(lax.rem(j + 2, NBUF), t + 2)
        return 0

    lax.fori_loop(0, NCHUNK + 1, g_body, 0)


def _make_sc_kernel():
    mesh = plsc.VectorSubcoreMesh(core_axis_name="c", subcore_axis_name="s")
    scratch = [
        pltpu.VMEM((NBUF, C, D), jnp.float32),
        pltpu.VMEM((NBUF, B, C, D), jnp.float32),
        pltpu.SemaphoreType.DMA((NBUF,)),
        pltpu.SemaphoreType.DMA((NBUF,)),
        pltpu.SemaphoreType.DMA((NBUF,)),
    ]
    return pl.kernel(
        _sc_body,
        out_type=jax.ShapeDtypeStruct((B, S, D), jnp.float32),
        mesh=mesh,
        scratch_types=scratch,
    )


def kernel(x, pos_table):
    return _make_sc_kernel()(x, pos_table)


# speedup vs baseline: 1.0462x; 1.0462x over previous
import jax
import jax.numpy as jnp
from jax import lax
from jax.experimental import pallas as pl
from jax.experimental.pallas import tpu as pltpu
from jax.experimental.pallas import tpu_sc as plsc

B, S, D = 4, 8192, 1024
NC, NS, L = 2, 16, 16
NW = NC * NS
ROWS_PER_W = S // NW
C = 8
NCHUNK = ROWS_PER_W // C
NBUF = 3

H = C // 2


def _sc_body(x_hbm, pos_hbm, out_hbm, posb, xb, insemsA, insemsB, outsems):
    cid = lax.axis_index("c")
    sid = lax.axis_index("s")
    wid = sid * NC + cid
    s0 = wid * ROWS_PER_W

    def in_descs(j, chunk, h, sems):
        s = s0 + chunk * C + h * H
        r = h * H
        cps = [pltpu.make_async_copy(
            pos_hbm.at[pl.ds(s, H), :], posb.at[j, pl.ds(r, H), :],
            sems.at[j])]
        for b in range(B):
            cps.append(pltpu.make_async_copy(
                x_hbm.at[b, pl.ds(s, H), :], xb.at[j, b, pl.ds(r, H), :],
                sems.at[j]))
        return cps

    def out_descs(j, chunk):
        s = s0 + chunk * C
        return [pltpu.make_async_copy(
            xb.at[j, b], out_hbm.at[b, pl.ds(s, C), :], outsems.at[j])
            for b in range(B)]

    def start_in(j, chunk):
        for cp in in_descs(j, chunk, 0, insemsA):
            cp.start()
        for cp in in_descs(j, chunk, 1, insemsB):
            cp.start()

    def wait_in(j, chunk, h, sems):
        for cp in in_descs(j, chunk, h, sems):
            cp.wait()

    def start_out(j, chunk):
        for cp in out_descs(j, chunk):
            cp.start()

    def wait_out(j, chunk):
        for cp in out_descs(j, chunk):
            cp.wait()

    def compute(j, h):
        for r in range(h * H, h * H + H):
            def col_body(cc, _, r=r):
                base = cc * (4 * L)
                for u in range(4):
                    off = base + u * L
                    p = posb[j, r, pl.ds(off, L)]
                    for b in range(B):
                        plsc.addupdate(xb.at[j, b, r, pl.ds(off, L)], p)
                return 0
            lax.fori_loop(0, D // (4 * L), col_body, 0, unroll=2)

    start_in(0, 0)
    start_in(1, 1)

    def g_body(t, _):
        j = lax.rem(t, NBUF)

        @pl.when(t < NCHUNK)
        def _():
            wait_in(j, t, 0, insemsA)
            compute(j, 0)

        @pl.when(t >= 1)
        def _():
            wait_out(lax.rem(j + NBUF - 1, NBUF), t - 1)

        @pl.when(t + 2 < NCHUNK)
        def _():
            start_in(lax.rem(j + 2, NBUF), t + 2)

        @pl.when(t < NCHUNK)
        def _():
            wait_in(j, t, 1, insemsB)
            compute(j, 1)
            start_out(j, t)
        return 0

    lax.fori_loop(0, NCHUNK + 1, g_body, 0)


def _make_sc_kernel():
    mesh = plsc.VectorSubcoreMesh(core_axis_name="c", subcore_axis_name="s")
    scratch = [
        pltpu.VMEM((NBUF, C, D), jnp.float32),
        pltpu.VMEM((NBUF, B, C, D), jnp.float32),
        pltpu.SemaphoreType.DMA((NBUF,)),
        pltpu.SemaphoreType.DMA((NBUF,)),
        pltpu.SemaphoreType.DMA((NBUF,)),
    ]
    return pl.kernel(
        _sc_body,
        out_type=jax.ShapeDtypeStruct((B, S, D), jnp.float32),
        mesh=mesh,
        scratch_types=scratch,
    )


def kernel(x, pos_table):
    return _make_sc_kernel()(x, pos_table)
